# baseline (speedup 1.0000x reference)
import jax
import jax.numpy as jnp
from jax.experimental import pallas as pl
from jax.experimental.pallas import tpu as pltpu

_N = 4096
_D = 1024
_T = 512
_NT = _N // _T
_H = 256
_EPS = 1e-8
_DIMNUMS = (((1,), (1,)), ((), ()))


def _koleo_kernel(x_ref, out_ref, xm2_ref, sqr_ref, sqc_ref, rowacc_ref,
                  colacc_ref):
    g = pl.program_id(0)

    @pl.when(g == 0)
    def _init():
        rowacc_ref[:] = jnp.full((_N, 128), jnp.inf, jnp.float32)
        colacc_ref[:] = jnp.full((1, _N), jnp.inf, jnp.float32)

    xrow = x_ref[:]
    rb = pl.ds(g * _T, _T)
    xgb = xrow.astype(jnp.bfloat16)
    xgm2 = (-2.0 * xrow).astype(jnp.bfloat16)
    xm2_ref[rb, :] = xgm2
    sq = jnp.sum(xrow * xrow, axis=1, keepdims=True)
    sqcv = sq.reshape(1, _T)
    sqr_ref[rb, :] = sq
    sqc_ref[:, rb] = sqcv

    def _lane_fold(t):
        return jnp.minimum(jnp.minimum(t[:, 0:128], t[:, 128:256]),
                           jnp.minimum(t[:, 256:384], t[:, 384:512]))

    for i in range(_NT - 1):
        @pl.when(i < g)
        def _off(i=i):
            xm2i = xm2_ref[i * _T:(i + 1) * _T, :]
            gp = jax.lax.dot_general(
                xgb, xm2i, _DIMNUMS,
                preferred_element_type=jnp.float32)
            t1 = sqc_ref[:, i * _T:(i + 1) * _T] + gp
            rowacc_ref[rb, :] = jnp.minimum(rowacc_ref[rb, :], _lane_fold(t1))
            t2 = sq + gp
            cb = pl.ds(i * _T, _T)
            colacc_ref[:, cb] = jnp.minimum(
                colacc_ref[:, cb], jnp.min(t2, axis=0, keepdims=True))

    for a, b in ((0, 0), (1, 0), (1, 1)):
        xga = xgb[a * _H:(a + 1) * _H, :]
        xm2b = xgm2[b * _H:(b + 1) * _H, :]
        gp = jax.lax.dot_general(
            xga, xm2b, _DIMNUMS, preferred_element_type=jnp.float32)
        t1 = sqcv[:, b * _H:(b + 1) * _H] + gp
        if a == b:
            rr = jax.lax.broadcasted_iota(jnp.int32, (_H, _H), 0)
            cc = jax.lax.broadcasted_iota(jnp.int32, (_H, _H), 1)
            t1 = jnp.where(rr == cc, jnp.inf, t1)
        ra = pl.ds(g * _T + a * _H, _H)
        rowacc_ref[ra, :] = jnp.minimum(
            rowacc_ref[ra, :], jnp.minimum(t1[:, 0:128], t1[:, 128:256]))
        if a != b:
            t2 = sq[a * _H:(a + 1) * _H, :] + gp
            cb = pl.ds(g * _T + b * _H, _H)
            colacc_ref[:, cb] = jnp.minimum(
                colacc_ref[:, cb], jnp.min(t2, axis=0, keepdims=True))

    @pl.when(g == _NT - 1)
    def _fin():
        rowmin = jnp.min(rowacc_ref[:], axis=1, keepdims=True)
        rowfull = rowmin + sqr_ref[:]
        colfull = (colacc_ref[:] + sqc_ref[:]).reshape(_N, 1)
        md2 = jnp.maximum(jnp.minimum(rowfull, colfull), 0.0)
        md2w = md2.reshape(32, 128)
        s = jnp.sum(jnp.log(jnp.sqrt(md2w) + _EPS), keepdims=True)
        out_ref[:, :] = s[0:1, 0:1] * (-1.0 / _N)


def kernel(student_output):
    out = pl.pallas_call(
        _koleo_kernel,
        grid=(_NT,),
        in_specs=[pl.BlockSpec((_T, _D), lambda g: (g, 0))],
        out_specs=pl.BlockSpec((1, 1), lambda g: (0, 0)),
        out_shape=jax.ShapeDtypeStruct((1, 1), jnp.float32),
        scratch_shapes=[
            pltpu.VMEM((_N, _D), jnp.bfloat16),
            pltpu.VMEM((_N, 1), jnp.float32),
            pltpu.VMEM((1, _N), jnp.float32),
            pltpu.VMEM((_N, 128), jnp.float32),
            pltpu.VMEM((1, _N), jnp.float32),
        ],
    )(student_output)
    return out[0, 0]
